# baseline (speedup 1.0000x reference)
import jax
import jax.numpy as jnp
from jax.experimental import pallas as pl
from jax.experimental.pallas import tpu as pltpu

_LANE = 128
_BLOCK = 32768


def kernel(input_ids, scores, allowed_ids):
    del input_ids
    batch, vocab = scores.shape
    nids = allowed_ids.shape[0]
    num_blocks = pl.cdiv(vocab, _BLOCK)

    def body(*refs):
        ids_ref = refs[0]
        score_refs = refs[1:1 + nids]
        out_ref = refs[1 + nids]
        i = pl.program_id(0)
        base = i * _BLOCK
        out_ref[...] = jnp.full((batch, _BLOCK), -jnp.inf, out_ref.dtype)
        coliota = jax.lax.broadcasted_iota(jnp.int32, (batch, _BLOCK), 1)
        laneiota = jax.lax.broadcasted_iota(jnp.int32, (batch, _LANE), 1)
        for j in range(nids):
            pos = ids_ref[j] - base

            @pl.when((pos >= 0) & (pos < _BLOCK))
            def _place(j=j, pos=pos):
                c = ids_ref[j] % _LANE
                col = jnp.sum(
                    jnp.where(laneiota == c, score_refs[j][...], 0.0),
                    axis=1, keepdims=True)
                out_ref[...] = jnp.where(coliota == pos, col, out_ref[...])

    in_specs = [
        pl.BlockSpec((batch, _LANE), (lambda i, ids, j=j: (0, ids[j] // _LANE)))
        for j in range(nids)
    ]
    out = pl.pallas_call(
        body,
        grid_spec=pltpu.PrefetchScalarGridSpec(
            num_scalar_prefetch=1,
            grid=(num_blocks,),
            in_specs=in_specs,
            out_specs=pl.BlockSpec((batch, _BLOCK), lambda i, ids: (0, i)),
        ),
        out_shape=jax.ShapeDtypeStruct((batch, vocab), scores.dtype),
    )(allowed_ids, *([scores] * nids))
    return out
